# baseline (speedup 1.0000x reference)
import functools

import jax
import jax.numpy as jnp
from jax import lax
from jax.experimental import pallas as pl
from jax.experimental.pallas import tpu as pltpu
from jax.experimental.pallas import tpu_sc as plsc

B = 4096
S = 200
D = 32
V = 1000000
NC = 2
NS = 16
NW = NC * NS
BW = B // NW

_mesh = plsc.VectorSubcoreMesh(core_axis_name="c", subcore_axis_name="s")


@functools.partial(
    pl.kernel,
    mesh=_mesh,
    compiler_params=pltpu.CompilerParams(
        use_tc_tiling_on_sc=False, needs_layout_passes=False),
    out_type=jax.ShapeDtypeStruct((S, 128, 1024), jnp.float32),
    scratch_types=[
        pltpu.VMEM((S, BW), jnp.int32),
        pltpu.VMEM((BW, D), jnp.float32),
        pltpu.VMEM((BW, D), jnp.float32),
        pltpu.VMEM((4096,), jnp.float32),
        pltpu.VMEM((4096,), jnp.float32),
        pltpu.VMEM((S, D), jnp.float32),
        pltpu.SemaphoreType.DMA,
        pltpu.SemaphoreType.DMA,
        pltpu.SemaphoreType.DMA,
        pltpu.SemaphoreType.DMA,
    ],
)
def _embed(xT_hbm, tok_hbm, pos_hbm, out_hbm,
           idxT, rows0, rows1, t0, t1, pos_v,
           gsem0, gsem1, ssem0, ssem1):
    wid = lax.axis_index("s") * NC + lax.axis_index("c")

    rows = (rows0, rows1)
    tt = (t0, t1)
    gsem = (gsem0, gsem1)
    ssem = (ssem0, ssem1)

    pltpu.sync_copy(xT_hbm.at[:, pl.ds(wid * BW, BW)], idxT)
    pltpu.sync_copy(pos_hbm, pos_v)

    def start_gather(s, b):
        pltpu.async_copy(tok_hbm.at[idxT.at[s]], rows[b], gsem[b])

    def wait_gather(b):
        pltpu.make_async_copy(
            tok_hbm.at[pl.ds(0, BW)], rows[b], gsem[b]).wait()

    def start_scatter(s, b):
        for g in range(4):
            pltpu.async_copy(
                tt[b].at[pl.ds(g * 1024, 1024)],
                out_hbm.at[s].at[g * 32 + wid], ssem[b])

    def wait_scatter(b):
        for g in range(4):
            pltpu.make_async_copy(
                tt[b].at[pl.ds(g * 1024, 1024)],
                out_hbm.at[0].at[0], ssem[b]).wait()

    def compute(s, b):
        rv = rows[b]
        tv = tt[b]

        def body(j, _):
            iot = lax.iota(jnp.int32, 16)
            sidx_lo = (iot // 8) * 1024 + (iot % 8) * 128 + j
            lo = rv[j, pl.ds(0, 16)] + pos_v[s, pl.ds(0, 16)]
            hi = rv[j, pl.ds(16, 16)] + pos_v[s, pl.ds(16, 16)]
            plsc.store_scatter(tv, [sidx_lo], lo)
            plsc.store_scatter(tv, [sidx_lo + 2 * 1024], hi)
            return 0

        lax.fori_loop(0, BW, body, 0)

    start_gather(0, 0)
    start_gather(1, 1)

    @pl.loop(0, S, step=2)
    def _planes(s0):
        for b in range(2):
            s = s0 + b

            @pl.when(s0 >= 2)
            def _():
                wait_scatter(b)

            wait_gather(b)
            compute(s, b)
            start_scatter(s, b)

            @pl.when(s0 < S - 2)
            def _():
                start_gather(s + 2, b)

    wait_scatter(0)
    wait_scatter(1)


def kernel(x, token_table, pos_table):
    xT = x.T.astype(jnp.int32)
    out = _embed(xT, token_table, pos_table)
    z = out.reshape(S, 4, D, 8, 128)
    z = z.transpose(2, 4, 0, 1, 3)
    return z.reshape(B, S, D)


# speedup vs baseline: 1.0351x; 1.0351x over previous
import functools

import jax
import jax.numpy as jnp
from jax import lax
from jax.experimental import pallas as pl
from jax.experimental.pallas import tpu as pltpu
from jax.experimental.pallas import tpu_sc as plsc

B = 4096
S = 200
D = 32
V = 1000000
NC = 2
NS = 16
NW = NC * NS
BW = B // NW

_mesh = plsc.VectorSubcoreMesh(core_axis_name="c", subcore_axis_name="s")


@functools.partial(
    pl.kernel,
    mesh=_mesh,
    compiler_params=pltpu.CompilerParams(
        use_tc_tiling_on_sc=False, needs_layout_passes=False),
    out_type=jax.ShapeDtypeStruct((S, 128, 1024), jnp.float32),
    scratch_types=[
        pltpu.VMEM((S, BW), jnp.int32),
        pltpu.VMEM((BW, D), jnp.float32),
        pltpu.VMEM((BW, D), jnp.float32),
        pltpu.VMEM((4096,), jnp.float32),
        pltpu.VMEM((4096,), jnp.float32),
        pltpu.VMEM((S, D), jnp.float32),
        pltpu.SemaphoreType.DMA,
        pltpu.SemaphoreType.DMA,
        pltpu.SemaphoreType.DMA,
        pltpu.SemaphoreType.DMA,
    ],
)
def _embed(xT_hbm, tok_hbm, pos_hbm, out_hbm,
           idxT, rows0, rows1, t0, t1, pos_v,
           gsem0, gsem1, ssem0, ssem1):
    wid = lax.axis_index("s") * NC + lax.axis_index("c")

    rows = (rows0, rows1)
    tt = (t0, t1)
    gsem = (gsem0, gsem1)
    ssem = (ssem0, ssem1)

    pltpu.sync_copy(xT_hbm.at[:, pl.ds(wid * BW, BW)], idxT)
    pltpu.sync_copy(pos_hbm, pos_v)

    def start_gather(s, b):
        pltpu.async_copy(tok_hbm.at[idxT.at[s]], rows[b], gsem[b])

    def wait_gather(b):
        pltpu.make_async_copy(
            tok_hbm.at[pl.ds(0, BW)], rows[b], gsem[b]).wait()

    def start_scatter(s, b):
        for g in range(4):
            pltpu.async_copy(
                tt[b].at[pl.ds(g * 1024, 1024)],
                out_hbm.at[s].at[g * 32 + wid], ssem[b])

    def wait_scatter(b):
        for g in range(4):
            pltpu.make_async_copy(
                tt[b].at[pl.ds(g * 1024, 1024)],
                out_hbm.at[0].at[0], ssem[b]).wait()

    def compute(s, b):
        rv = rows[b]
        tv = tt[b]
        iot = lax.iota(jnp.int32, 16)
        base_lo = (iot // 8) * 1024 + (iot % 8) * 128
        p_lo = pos_v[s, pl.ds(0, 16)]
        p_hi = pos_v[s, pl.ds(16, 16)]

        def body(j8, _):
            jb = j8 * 8
            for u in range(8):
                j = jb + u
                sidx_lo = base_lo + j
                lo = rv[j, pl.ds(0, 16)] + p_lo
                hi = rv[j, pl.ds(16, 16)] + p_hi
                plsc.store_scatter(tv, [sidx_lo], lo)
                plsc.store_scatter(tv, [sidx_lo + 2 * 1024], hi)
            return 0

        lax.fori_loop(0, BW // 8, body, 0)

    start_gather(0, 0)
    start_gather(1, 1)

    @pl.loop(0, S, step=2)
    def _planes(s0):
        for b in range(2):
            s = s0 + b

            @pl.when(s0 >= 2)
            def _():
                wait_scatter(b)

            wait_gather(b)
            compute(s, b)
            start_scatter(s, b)

            @pl.when(s0 < S - 2)
            def _():
                start_gather(s + 2, b)

    wait_scatter(0)
    wait_scatter(1)


def kernel(x, token_table, pos_table):
    xT = x.T.astype(jnp.int32)
    out = _embed(xT, token_table, pos_table)
    z = out.reshape(S, 4, D, 8, 128)
    z = z.transpose(2, 4, 0, 1, 3)
    return z.reshape(B, S, D)
